# baseline (speedup 1.0000x reference)
import functools

import jax
import jax.numpy as jnp
import numpy as np
from jax import lax
from jax.experimental import pallas as pl
from jax.experimental.pallas import tpu as pltpu
from jax.experimental.pallas import tpu_sc as plsc

_FIELD_DIMS = [100000] * 26
_OFFS = np.array((0, *np.cumsum(_FIELD_DIMS)[:-1]), dtype=np.int32)
_B = 4096
_NF = 26
_D = 16
_EPS = 1e-5

_NC = 2
_NS = 16
_NW = _NC * _NS
_RPW = _B // _NW
_EPW = _RPW * _NF * _D
_LPW = _RPW * _NF
_CHUNK = 128
_NCH = _EPW // _CHUNK


def _sc_gather_body(idx_hbm, emb_hbm, lin_hbm, emb_out, lin_out,
                    idx_v, list_v, ebuf, lbuf, sem_e, sem_l):
    wid = lax.axis_index("s") * _NC + lax.axis_index("c")
    cbase = pl.multiple_of(wid * _RPW, 8)
    pltpu.sync_copy(idx_hbm.at[:, pl.ds(cbase, _RPW)], idx_v)

    iota = lax.iota(jnp.int32, 16)
    v_rows = _NF * _FIELD_DIMS[0]

    def build_fg(k, carry):
        f = k // 8
        g = k - f * 8
        w = idx_v[f, pl.ds(g * 16, 16)]
        pos0 = iota * (_NF * _D) + (g * 16 * _NF + f) * _D
        for d in range(_D):
            plsc.store_scatter(list_v, [pos0 + d], w + d * v_rows)
        return carry

    lax.fori_loop(0, _NF * 8, build_fg, 0)

    def fire(t, carry):
        off = pl.multiple_of(t * _CHUNK, 8)
        pltpu.make_async_copy(
            emb_hbm.at[list_v.at[pl.ds(off, _CHUNK)]],
            ebuf.at[pl.ds(off, _CHUNK)],
            sem_e,
        ).start()
        return carry

    lax.fori_loop(0, _NCH, fire, 0)

    for f in range(_NF):
        pltpu.make_async_copy(
            lin_hbm.at[idx_v.at[f]],
            lbuf.at[pl.ds(f * _RPW, _RPW)],
            sem_l,
        ).start()

    pltpu.make_async_copy(emb_hbm.at[pl.ds(0, _EPW)], ebuf, sem_e).wait()
    pltpu.make_async_copy(lin_hbm.at[pl.ds(0, _LPW)], lbuf, sem_l).wait()

    ebase = pl.multiple_of(wid * _EPW, 8)
    lbase = pl.multiple_of(wid * _LPW, 8)
    pltpu.sync_copy(ebuf, emb_out.at[pl.ds(ebase, _EPW)])
    pltpu.sync_copy(lbuf, lin_out.at[pl.ds(lbase, _LPW)])


@functools.lru_cache(maxsize=1)
def _make_sc_gather():
    mesh = plsc.VectorSubcoreMesh(
        core_axis_name="c", subcore_axis_name="s",
        num_cores=_NC, num_subcores=_NS,
    )
    return pl.kernel(
        _sc_gather_body,
        out_type=(
            jax.ShapeDtypeStruct((_NW * _EPW,), jnp.float32),
            jax.ShapeDtypeStruct((_NW * _LPW,), jnp.float32),
        ),
        mesh=mesh,
        compiler_params=pltpu.CompilerParams(needs_layout_passes=False),
        scratch_types=[
            pltpu.VMEM((_NF, _RPW), jnp.int32),
            pltpu.VMEM((_EPW,), jnp.int32),
            pltpu.VMEM((_EPW,), jnp.float32),
            pltpu.VMEM((_LPW,), jnp.float32),
            pltpu.SemaphoreType.DMA,
            pltpu.SemaphoreType.DMA,
        ],
    )


_RB = 512


def _tc_head(emb_ref, linv_ref, w0_ref, b0_ref, w1_ref, b1_ref, wo_ref,
             a_ref, d_ref):
    e = emb_ref[...]
    s = e[:, 0:_D]
    q = s * s
    for f in range(1, _NF):
        c = e[:, f * _D:(f + 1) * _D]
        s = s + c
        q = q + c * c
    inter = 0.5 * jnp.sum(s * s - q, axis=1, keepdims=True)
    lin = jnp.sum(linv_ref[...], axis=1, keepdims=True)
    a_ref[...] = lin + inter
    h = jnp.dot(e, w0_ref[...], preferred_element_type=jnp.float32)
    h = jnp.maximum(h + b0_ref[...], 0.0)
    h = jnp.dot(h, w1_ref[...], preferred_element_type=jnp.float32)
    h = jnp.maximum(h + b1_ref[...], 0.0)
    d_ref[...] = jnp.dot(h, wo_ref[...], preferred_element_type=jnp.float32)


def _tc_out(a_ref, dt_ref, out_ref):
    out_ref[...] = jax.nn.sigmoid(a_ref[...] + dt_ref[...])


def kernel(x, W_emb, W_lin, lin_bias, W0, b0, g0, be0, rm0, rv0,
           W1, b1, g1, be1, rm1, rv1, Wout, bout):
    xi2 = x.T + jnp.asarray(_OFFS, dtype=x.dtype)[:, None]
    emb1 = W_emb.T.reshape(-1)
    lin1 = W_lin.reshape(-1)

    inv0 = g0 / jnp.sqrt(rv0 + _EPS)
    w0t = (W0 * inv0[:, None]).T
    b0f = ((b0 - rm0) * inv0 + be0).reshape(1, -1)
    inv1 = g1 / jnp.sqrt(rv1 + _EPS)
    w1t = (W1 * inv1[:, None]).T
    b1f = ((b1 - rm1) * inv1 + be1).reshape(1, -1)
    wot = Wout.T
    bias_all = (bout + lin_bias)[0]

    emb_flat, lin_flat = _make_sc_gather()(xi2, emb1, lin1)
    emb2d = emb_flat.reshape(_B, _NF * _D)
    linv = (lin_flat.reshape(_NW, _NF, _RPW)
            .transpose(0, 2, 1).reshape(_B, _NF))

    n_blk = _B // _RB
    a, d = pl.pallas_call(
        _tc_head,
        grid=(n_blk,),
        in_specs=[
            pl.BlockSpec((_RB, _NF * _D), lambda i: (i, 0)),
            pl.BlockSpec((_RB, _NF), lambda i: (i, 0)),
            pl.BlockSpec((_NF * _D, 128), lambda i: (0, 0)),
            pl.BlockSpec((1, 128), lambda i: (0, 0)),
            pl.BlockSpec((128, 64), lambda i: (0, 0)),
            pl.BlockSpec((1, 64), lambda i: (0, 0)),
            pl.BlockSpec((64, 1), lambda i: (0, 0)),
        ],
        out_specs=[
            pl.BlockSpec((_RB, 1), lambda i: (i, 0)),
            pl.BlockSpec((_RB, 1), lambda i: (i, 0)),
        ],
        out_shape=[
            jax.ShapeDtypeStruct((_B, 1), jnp.float32),
            jax.ShapeDtypeStruct((_B, 1), jnp.float32),
        ],
    )(emb2d, linv, w0t, b0f, w1t, b1f, wot)

    dt = (d[:, 0] + bias_all).reshape(1, _B)

    out = pl.pallas_call(
        _tc_out,
        grid=(n_blk,),
        in_specs=[
            pl.BlockSpec((_RB, 1), lambda i: (i, 0)),
            pl.BlockSpec((1, _B), lambda i: (0, 0)),
        ],
        out_specs=pl.BlockSpec((_RB, _B), lambda i: (i, 0)),
        out_shape=jax.ShapeDtypeStruct((_B, _B), jnp.float32),
    )(a, dt)
    return out
